# baseline (speedup 1.0000x reference)
import jax
import jax.numpy as jnp
from jax import lax
from jax.experimental import pallas as pl
from jax.experimental.pallas import tpu as pltpu
from jax.experimental.pallas import tpu_sc as plsc

NN = 10000
EE = 320000
DD = 128
HH = 32
NC = 2
NS = 16
NW = NC * NS
NP = 10240
RPT = NP // NS
CH = 128
NCHUNK = 80
EPT = CH * NCHUNK
EP = EPT * NW

_f32 = jnp.float32
_i32 = jnp.int32


def _sc_mesh():
    return plsc.VectorSubcoreMesh(core_axis_name="c", subcore_axis_name="s")


def _sc_params():
    return pltpu.CompilerParams(use_tc_tiling_on_sc=False)


def _deg_body(dstg, ones16, zeros16, deg_out, dst_v, ones_v, buf_v,
              sem_a, sem_b, sem_c, sem_s, deg_sh):
    c = lax.axis_index("c")
    s = lax.axis_index("s")
    w = s * NC + c
    rs = s * RPT
    a1 = pltpu.async_copy(dstg.at[w], dst_v, sem_a)
    a2 = pltpu.async_copy(ones16, ones_v, sem_b)
    a3 = pltpu.async_copy(zeros16, buf_v, sem_c)
    a3.wait()
    b3 = pltpu.async_copy(buf_v, deg_sh.at[pl.ds(rs, RPT)], sem_c)
    a1.wait()
    a2.wait()
    b3.wait()
    plsc.subcore_barrier()

    def group(j, carry):
        for k in range(8):
            pltpu.async_copy(ones_v, deg_sh.at[dst_v.at[j * 8 + k]], sem_s,
                             add=True)
        for k in range(8):
            pltpu.make_async_copy(ones_v, deg_sh.at[dst_v.at[j * 8 + k]],
                                  sem_s).wait()
        return carry

    lax.fori_loop(0, NCHUNK // 8, group, 0)
    plsc.subcore_barrier()
    pltpu.sync_copy(deg_sh.at[pl.ds(rs, RPT)], buf_v)
    pltpu.sync_copy(buf_v, deg_out.at[c, pl.ds(rs, RPT)])


def _deg_call(dstg, ones16, zeros16):
    fn = pl.kernel(
        _deg_body,
        out_type=jax.ShapeDtypeStruct((NC, NP, 16), _f32),
        mesh=_sc_mesh(),
        compiler_params=_sc_params(),
        scratch_types=[
            pltpu.VMEM((NCHUNK, CH), _i32),
            pltpu.VMEM((CH, 16), _f32),
            pltpu.VMEM((RPT, 16), _f32),
            pltpu.SemaphoreType.DMA,
            pltpu.SemaphoreType.DMA,
            pltpu.SemaphoreType.DMA,
            pltpu.SemaphoreType.DMA,
            pltpu.VMEM_SHARED((NP, 16), _f32),
        ],
    )
    return fn(dstg, ones16, zeros16)


def _prop_body(hws, srcg, dstg, zeros32, z_out, src_v, dst_v, rows_v, buf_v,
               buf2_v, sem_g, sem_s, sem_a, sem_b, sem_d, z_sh):
    c = lax.axis_index("c")
    s = lax.axis_index("s")
    w = s * NC + c
    rs = s * RPT
    a1 = pltpu.async_copy(srcg.at[w], src_v, sem_a)
    a2 = pltpu.async_copy(dstg.at[w], dst_v, sem_b)
    a4 = pltpu.async_copy(zeros32, buf2_v, sem_d)
    a4.wait()
    b4 = pltpu.async_copy(buf2_v, z_sh.at[pl.ds(rs, RPT)], sem_d)
    a1.wait()
    a2.wait()
    b4.wait()
    plsc.subcore_barrier()

    pltpu.async_copy(hws.at[src_v.at[0]], rows_v.at[pl.ds(0, CH)], sem_g)

    def chunk(i, carry):
        off = (i & 1) * CH
        noff = ((i + 1) & 1) * CH
        pltpu.make_async_copy(hws.at[src_v.at[i]],
                              rows_v.at[pl.ds(off, CH)], sem_g).wait()
        pltpu.async_copy(rows_v.at[pl.ds(off, CH)], z_sh.at[dst_v.at[i]],
                         sem_s, add=True)

        @pl.when(i + 1 < NCHUNK)
        def _():
            @pl.when(i >= 1)
            def _():
                pltpu.make_async_copy(rows_v.at[pl.ds(noff, CH)],
                                      z_sh.at[dst_v.at[i - 1]], sem_s).wait()
            pltpu.async_copy(hws.at[src_v.at[i + 1]],
                             rows_v.at[pl.ds(noff, CH)], sem_g)

        return carry

    lax.fori_loop(0, NCHUNK, chunk, 0)
    pltpu.make_async_copy(rows_v.at[pl.ds(0, CH)],
                          z_sh.at[dst_v.at[NCHUNK - 2]], sem_s).wait()
    pltpu.make_async_copy(rows_v.at[pl.ds(0, CH)],
                          z_sh.at[dst_v.at[NCHUNK - 1]], sem_s).wait()
    plsc.subcore_barrier()
    pltpu.sync_copy(z_sh.at[pl.ds(rs, RPT)], buf_v)
    pltpu.sync_copy(buf_v, z_out.at[c, pl.ds(rs, RPT)])


def _prop_call(hws, srcg, dstg, zeros32):
    fn = pl.kernel(
        _prop_body,
        out_type=jax.ShapeDtypeStruct((NC, NP, HH), _f32),
        mesh=_sc_mesh(),
        compiler_params=_sc_params(),
        scratch_types=[
            pltpu.VMEM((NCHUNK, CH), _i32),
            pltpu.VMEM((NCHUNK, CH), _i32),
            pltpu.VMEM((2 * CH, HH), _f32),
            pltpu.VMEM((RPT, HH), _f32),
            pltpu.VMEM((RPT, HH), _f32),
            pltpu.SemaphoreType.DMA,
            pltpu.SemaphoreType.DMA,
            pltpu.SemaphoreType.DMA,
            pltpu.SemaphoreType.DMA,
            pltpu.SemaphoreType.DMA,
            pltpu.VMEM_SHARED((NP, HH), _f32),
        ],
    )
    return fn(hws, srcg, dstg, zeros32)


def _tc1a_body(x_ref, w_ref, hw_ref):
    hw_ref[...] = jnp.dot(x_ref[...], w_ref[...], preferred_element_type=_f32)


def _tc1a_call(xp, W1):
    return pl.pallas_call(
        _tc1a_body,
        out_shape=jax.ShapeDtypeStruct((NP, HH), _f32),
    )(xp, W1)


def _tc1b_body(degp_ref, hw_ref, dis_ref, hws_ref):
    deg = degp_ref[0, :, 0:1] + degp_ref[1, :, 0:1] + 1.0
    dis = lax.rsqrt(deg)
    dis_ref[...] = dis
    hws_ref[...] = hw_ref[...] * dis


def _tc1b_call(degp, hw1):
    return pl.pallas_call(
        _tc1b_body,
        out_shape=(
            jax.ShapeDtypeStruct((NP, 1), _f32),
            jax.ShapeDtypeStruct((NP, HH), _f32),
        ),
    )(degp, hw1)


def _tc_mid_body(zp_ref, hws_ref, dis_ref, b_ref, w_ref, out_ref):
    agg = zp_ref[0] + zp_ref[1] + hws_ref[...]
    dis = dis_ref[...]
    h = jnp.maximum(agg * dis + b_ref[...], 0.0)
    out_ref[...] = jnp.dot(h, w_ref[...], preferred_element_type=_f32) * dis


def _tc_mid_call(zp, hws, dis, b, W):
    return pl.pallas_call(
        _tc_mid_body,
        out_shape=jax.ShapeDtypeStruct((NP, HH), _f32),
    )(zp, hws, dis, b, W)


def _tc4_body(zp_ref, hws_ref, dis_ref, b3_ref, m1w_ref, m1b_ref, m2w_ref,
              m2b_ref, out_ref):
    agg = zp_ref[0] + zp_ref[1] + hws_ref[...]
    h = jnp.maximum(agg * dis_ref[...] + b3_ref[...], 0.0)
    h = jnp.maximum(jnp.dot(h, m1w_ref[...], preferred_element_type=_f32)
                    + m1b_ref[...], 0.0)
    h = jnp.maximum(jnp.dot(h, m2w_ref[...], preferred_element_type=_f32)
                    + m2b_ref[...], 0.0)
    out_ref[...] = h[:NN]


def _tc4_call(zp, hws, dis, b3, M1W, M1b, M2W, M2b):
    return pl.pallas_call(
        _tc4_body,
        out_shape=jax.ShapeDtypeStruct((NN, HH), _f32),
    )(zp, hws, dis, b3, M1W, M1b, M2W, M2b)


def kernel(x, edge_index, W1, b1, W2, b2, W3, b3, M1W, M1b, M2W, M2b):
    src = edge_index[0]
    dst = edge_index[1]
    srcg = jnp.concatenate(
        [src, jnp.zeros((EP - EE,), _i32)]).reshape(NW, NCHUNK, CH)
    dstg = jnp.concatenate(
        [dst, jnp.full((EP - EE,), NN, _i32)]).reshape(NW, NCHUNK, CH)
    ones16 = jnp.ones((CH, 16), _f32)
    zeros16 = jnp.zeros((RPT, 16), _f32)
    zeros32 = jnp.zeros((RPT, HH), _f32)
    xp = jnp.pad(x, ((0, NP - NN), (0, 0)))

    hw1 = _tc1a_call(xp, W1)
    degp = _deg_call(dstg, ones16, zeros16)
    dis, hws1 = _tc1b_call(degp, hw1)
    z1 = _prop_call(hws1, srcg, dstg, zeros32)
    hws2 = _tc_mid_call(z1, hws1, dis, b1.reshape(1, HH), W2)
    z2 = _prop_call(hws2, srcg, dstg, zeros32)
    hws3 = _tc_mid_call(z2, hws2, dis, b2.reshape(1, HH), W3)
    z3 = _prop_call(hws3, srcg, dstg, zeros32)
    out = _tc4_call(z3, hws3, dis, b3.reshape(1, HH), M1W,
                    M1b.reshape(1, 64), M2W, M2b.reshape(1, HH))
    return out


# speedup vs baseline: 2.1297x; 2.1297x over previous
import jax
import jax.numpy as jnp
from jax import lax
from jax.experimental import pallas as pl
from jax.experimental.pallas import tpu as pltpu
from jax.experimental.pallas import tpu_sc as plsc

NN = 10000
EE = 320000
DD = 128
HH = 32
NC = 2
NS = 16
NW = NC * NS
NP = 10240
RPT = NP // NS
CH = 128
NCHUNK = 80
EPT = CH * NCHUNK
EP = EPT * NW

_f32 = jnp.float32
_i32 = jnp.int32


def _sc_mesh():
    return plsc.VectorSubcoreMesh(core_axis_name="c", subcore_axis_name="s")


def _sc_params():
    return pltpu.CompilerParams(use_tc_tiling_on_sc=False)


def _deg_body(dstg, ones16, zeros16, deg_out, dst_v, ones_v, buf_v,
              sem_a, sem_b, sem_c, sem_s, deg_sh):
    c = lax.axis_index("c")
    s = lax.axis_index("s")
    w = s * NC + c
    rs = s * RPT
    a1 = pltpu.async_copy(dstg.at[w], dst_v, sem_a)
    a2 = pltpu.async_copy(ones16, ones_v, sem_b)
    a3 = pltpu.async_copy(zeros16, buf_v, sem_c)
    a3.wait()
    b3 = pltpu.async_copy(buf_v, deg_sh.at[pl.ds(rs, RPT)], sem_c)
    a1.wait()
    a2.wait()
    b3.wait()
    plsc.subcore_barrier()

    def group(j, carry):
        for k in range(8):
            pltpu.async_copy(ones_v, deg_sh.at[dst_v.at[j * 8 + k]], sem_s,
                             add=True)
        for k in range(8):
            pltpu.make_async_copy(ones_v, deg_sh.at[dst_v.at[j * 8 + k]],
                                  sem_s).wait()
        return carry

    lax.fori_loop(0, NCHUNK // 8, group, 0)
    plsc.subcore_barrier()
    pltpu.sync_copy(deg_sh.at[pl.ds(rs, RPT)], buf_v)
    pltpu.sync_copy(buf_v, deg_out.at[c, pl.ds(rs, RPT)])


def _deg_call(dstg, ones16, zeros16):
    fn = pl.kernel(
        _deg_body,
        out_type=jax.ShapeDtypeStruct((NC, NP, 16), _f32),
        mesh=_sc_mesh(),
        compiler_params=_sc_params(),
        scratch_types=[
            pltpu.VMEM((NCHUNK, CH), _i32),
            pltpu.VMEM((CH, 16), _f32),
            pltpu.VMEM((RPT, 16), _f32),
            pltpu.SemaphoreType.DMA,
            pltpu.SemaphoreType.DMA,
            pltpu.SemaphoreType.DMA,
            pltpu.SemaphoreType.DMA,
            pltpu.VMEM_SHARED((NP, 16), _f32),
        ],
    )
    return fn(dstg, ones16, zeros16)


def _prop_body(hws, srcg, dstg, zeros32, z_out, src_v, dst_v, rows_v, buf_v,
               buf2_v, sem_g, sem_s, sem_a, sem_b, sem_c, sem_d,
               table_sh, z_sh):
    c = lax.axis_index("c")
    s = lax.axis_index("s")
    w = s * NC + c
    rs = s * RPT
    a1 = pltpu.async_copy(srcg.at[w], src_v, sem_a)
    a2 = pltpu.async_copy(dstg.at[w], dst_v, sem_b)
    a3 = pltpu.async_copy(hws.at[pl.ds(rs, RPT)], buf_v, sem_c)
    a4 = pltpu.async_copy(zeros32, buf2_v, sem_d)
    a3.wait()
    b3 = pltpu.async_copy(buf_v, table_sh.at[pl.ds(rs, RPT)], sem_c)
    a4.wait()
    b4 = pltpu.async_copy(buf2_v, z_sh.at[pl.ds(rs, RPT)], sem_d)
    a1.wait()
    a2.wait()
    b3.wait()
    b4.wait()
    plsc.subcore_barrier()

    pltpu.async_copy(table_sh.at[src_v.at[0]], rows_v.at[pl.ds(0, CH)], sem_g)
    pltpu.async_copy(table_sh.at[src_v.at[1]], rows_v.at[pl.ds(CH, CH)], sem_g)
    pltpu.async_copy(table_sh.at[src_v.at[2]],
                     rows_v.at[pl.ds(2 * CH, CH)], sem_g)

    def chunk(i, carry):
        off = (i & 3) * CH
        noff = ((i + 3) & 3) * CH
        pltpu.make_async_copy(table_sh.at[src_v.at[i]],
                              rows_v.at[pl.ds(off, CH)], sem_g).wait()
        pltpu.async_copy(rows_v.at[pl.ds(off, CH)], z_sh.at[dst_v.at[i]],
                         sem_s, add=True)

        @pl.when(i + 3 < NCHUNK)
        def _():
            @pl.when(i >= 1)
            def _():
                pltpu.make_async_copy(rows_v.at[pl.ds(noff, CH)],
                                      z_sh.at[dst_v.at[i - 1]], sem_s).wait()
            pltpu.async_copy(table_sh.at[src_v.at[i + 3]],
                             rows_v.at[pl.ds(noff, CH)], sem_g)

        return carry

    lax.fori_loop(0, NCHUNK, chunk, 0)
    for t in range(4):
        pltpu.make_async_copy(rows_v.at[pl.ds(0, CH)],
                              z_sh.at[dst_v.at[NCHUNK - 4 + t]], sem_s).wait()
    plsc.subcore_barrier()
    pltpu.sync_copy(z_sh.at[pl.ds(rs, RPT)], buf_v)
    pltpu.sync_copy(buf_v, z_out.at[c, pl.ds(rs, RPT)])


def _prop_call(hws, srcg, dstg, zeros32):
    fn = pl.kernel(
        _prop_body,
        out_type=jax.ShapeDtypeStruct((NC, NP, HH), _f32),
        mesh=_sc_mesh(),
        compiler_params=_sc_params(),
        scratch_types=[
            pltpu.VMEM((NCHUNK, CH), _i32),
            pltpu.VMEM((NCHUNK, CH), _i32),
            pltpu.VMEM((4 * CH, HH), _f32),
            pltpu.VMEM((RPT, HH), _f32),
            pltpu.VMEM((RPT, HH), _f32),
            pltpu.SemaphoreType.DMA,
            pltpu.SemaphoreType.DMA,
            pltpu.SemaphoreType.DMA,
            pltpu.SemaphoreType.DMA,
            pltpu.SemaphoreType.DMA,
            pltpu.SemaphoreType.DMA,
            pltpu.VMEM_SHARED((NP, HH), _f32),
            pltpu.VMEM_SHARED((NP, HH), _f32),
        ],
    )
    return fn(hws, srcg, dstg, zeros32)


def _tc1a_body(x_ref, w_ref, hw_ref):
    hw_ref[...] = jnp.dot(x_ref[...], w_ref[...], preferred_element_type=_f32)


def _tc1a_call(xp, W1):
    return pl.pallas_call(
        _tc1a_body,
        out_shape=jax.ShapeDtypeStruct((NP, HH), _f32),
    )(xp, W1)


def _tc1b_body(degp_ref, hw_ref, dis_ref, hws_ref):
    deg = degp_ref[0, :, 0:1] + degp_ref[1, :, 0:1] + 1.0
    dis = lax.rsqrt(deg)
    dis_ref[...] = dis
    hws_ref[...] = hw_ref[...] * dis


def _tc1b_call(degp, hw1):
    return pl.pallas_call(
        _tc1b_body,
        out_shape=(
            jax.ShapeDtypeStruct((NP, 1), _f32),
            jax.ShapeDtypeStruct((NP, HH), _f32),
        ),
    )(degp, hw1)


def _tc_mid_body(zp_ref, hws_ref, dis_ref, b_ref, w_ref, out_ref):
    agg = zp_ref[0] + zp_ref[1] + hws_ref[...]
    dis = dis_ref[...]
    h = jnp.maximum(agg * dis + b_ref[...], 0.0)
    out_ref[...] = jnp.dot(h, w_ref[...], preferred_element_type=_f32) * dis


def _tc_mid_call(zp, hws, dis, b, W):
    return pl.pallas_call(
        _tc_mid_body,
        out_shape=jax.ShapeDtypeStruct((NP, HH), _f32),
    )(zp, hws, dis, b, W)


def _tc4_body(zp_ref, hws_ref, dis_ref, b3_ref, m1w_ref, m1b_ref, m2w_ref,
              m2b_ref, out_ref):
    agg = zp_ref[0] + zp_ref[1] + hws_ref[...]
    h = jnp.maximum(agg * dis_ref[...] + b3_ref[...], 0.0)
    h = jnp.maximum(jnp.dot(h, m1w_ref[...], preferred_element_type=_f32)
                    + m1b_ref[...], 0.0)
    h = jnp.maximum(jnp.dot(h, m2w_ref[...], preferred_element_type=_f32)
                    + m2b_ref[...], 0.0)
    out_ref[...] = h[:NN]


def _tc4_call(zp, hws, dis, b3, M1W, M1b, M2W, M2b):
    return pl.pallas_call(
        _tc4_body,
        out_shape=jax.ShapeDtypeStruct((NN, HH), _f32),
    )(zp, hws, dis, b3, M1W, M1b, M2W, M2b)


def kernel(x, edge_index, W1, b1, W2, b2, W3, b3, M1W, M1b, M2W, M2b):
    src = edge_index[0]
    dst = edge_index[1]
    srcg = jnp.concatenate(
        [src, jnp.zeros((EP - EE,), _i32)]).reshape(NW, NCHUNK, CH)
    dstg = jnp.concatenate(
        [dst, jnp.full((EP - EE,), NN, _i32)]).reshape(NW, NCHUNK, CH)
    ones16 = jnp.ones((CH, 16), _f32)
    zeros16 = jnp.zeros((RPT, 16), _f32)
    zeros32 = jnp.zeros((RPT, HH), _f32)
    xp = jnp.pad(x, ((0, NP - NN), (0, 0)))

    hw1 = _tc1a_call(xp, W1)
    degp = _deg_call(dstg, ones16, zeros16)
    dis, hws1 = _tc1b_call(degp, hw1)
    z1 = _prop_call(hws1, srcg, dstg, zeros32)
    hws2 = _tc_mid_call(z1, hws1, dis, b1.reshape(1, HH), W2)
    z2 = _prop_call(hws2, srcg, dstg, zeros32)
    hws3 = _tc_mid_call(z2, hws2, dis, b2.reshape(1, HH), W3)
    z3 = _prop_call(hws3, srcg, dstg, zeros32)
    out = _tc4_call(z3, hws3, dis, b3.reshape(1, HH), M1W,
                    M1b.reshape(1, 64), M2W, M2b.reshape(1, HH))
    return out


# speedup vs baseline: 2.8774x; 1.3511x over previous
import jax
import jax.numpy as jnp
from jax import lax
from jax.experimental import pallas as pl
from jax.experimental.pallas import tpu as pltpu
from jax.experimental.pallas import tpu_sc as plsc

NN = 10000
EE = 320000
DD = 128
HH = 32
NC = 2
NS = 16
NW = NC * NS
NP = 10240
RPT = NP // NS
CH = 80
NCHUNK = 125
EPT = CH * NCHUNK
NW4 = NP // 4
ND8 = NP // 8

_f32 = jnp.float32
_i32 = jnp.int32


def _sc_mesh():
    return plsc.VectorSubcoreMesh(core_axis_name="c", subcore_axis_name="s")


def _sc_params():
    return pltpu.CompilerParams(use_tc_tiling_on_sc=False)


def _deg_body(dstg, ones32, zeros32, deg_out, dst_v, ones_v, buf_v,
              sem_a, sem_b, sem_c, sem_s, deg_sh):
    c = lax.axis_index("c")
    s = lax.axis_index("s")
    w = s * NC + c
    rs = s * RPT
    a1 = pltpu.async_copy(dstg.at[w], dst_v, sem_a)
    a2 = pltpu.async_copy(ones32, ones_v, sem_b)
    a3 = pltpu.async_copy(zeros32, buf_v, sem_c)
    a3.wait()
    b3 = pltpu.async_copy(buf_v, deg_sh.at[pl.ds(rs, RPT)], sem_c)
    a1.wait()
    a2.wait()
    b3.wait()
    plsc.subcore_barrier()

    def chunk(i, carry):
        pltpu.async_copy(ones_v, deg_sh.at[dst_v.at[i]], sem_s, add=True)

        @pl.when(i >= 8)
        def _():
            pltpu.make_async_copy(ones_v, deg_sh.at[dst_v.at[i - 8]],
                                  sem_s).wait()

        return carry

    lax.fori_loop(0, NCHUNK, chunk, 0)
    for t in range(8):
        pltpu.make_async_copy(ones_v, deg_sh.at[dst_v.at[NCHUNK - 8 + t]],
                              sem_s).wait()
    plsc.subcore_barrier()
    pltpu.sync_copy(deg_sh.at[pl.ds(rs, RPT)], buf_v)
    pltpu.sync_copy(buf_v, deg_out.at[c, pl.ds(rs, RPT)])


def _deg_call(dstg, ones32, zeros32):
    fn = pl.kernel(
        _deg_body,
        out_type=jax.ShapeDtypeStruct((NC, NP, HH), _f32),
        mesh=_sc_mesh(),
        compiler_params=_sc_params(),
        scratch_types=[
            pltpu.VMEM((NCHUNK, CH), _i32),
            pltpu.VMEM((CH, HH), _f32),
            pltpu.VMEM((RPT, HH), _f32),
            pltpu.SemaphoreType.DMA,
            pltpu.SemaphoreType.DMA,
            pltpu.SemaphoreType.DMA,
            pltpu.SemaphoreType.DMA,
            pltpu.VMEM_SHARED((NP, HH), _f32),
        ],
    )
    return fn(dstg, ones32, zeros32)


def _prop_body(hws, srcg, dstg, zeros32, z_out, src_v, dst_v, rows_v, buf_v,
               buf2_v, sem_g, sem_s, sem_a, sem_b, sem_c, sem_d,
               table_sh, z_sh):
    c = lax.axis_index("c")
    s = lax.axis_index("s")
    w = s * NC + c
    rs = s * RPT
    a1 = pltpu.async_copy(srcg.at[w], src_v, sem_a)
    a2 = pltpu.async_copy(dstg.at[w], dst_v, sem_b)
    a3 = pltpu.async_copy(hws.at[pl.ds(rs, RPT)], buf_v, sem_c)
    a4 = pltpu.async_copy(zeros32, buf2_v, sem_d)
    a3.wait()
    b3 = pltpu.async_copy(buf_v, table_sh.at[pl.ds(rs, RPT)], sem_c)
    a4.wait()
    b4 = pltpu.async_copy(buf2_v, z_sh.at[pl.ds(rs, RPT)], sem_d)
    a1.wait()
    a2.wait()
    b3.wait()
    b4.wait()
    plsc.subcore_barrier()

    pltpu.async_copy(table_sh.at[src_v.at[0]], rows_v.at[pl.ds(0, CH)], sem_g)
    pltpu.async_copy(table_sh.at[src_v.at[1]], rows_v.at[pl.ds(CH, CH)], sem_g)
    pltpu.async_copy(table_sh.at[src_v.at[2]],
                     rows_v.at[pl.ds(2 * CH, CH)], sem_g)

    def chunk(i, carry):
        off = (i & 3) * CH
        noff = ((i + 3) & 3) * CH
        pltpu.make_async_copy(table_sh.at[src_v.at[i]],
                              rows_v.at[pl.ds(off, CH)], sem_g).wait()
        pltpu.async_copy(rows_v.at[pl.ds(off, CH)], z_sh.at[dst_v.at[i]],
                         sem_s, add=True)

        @pl.when(i + 3 < NCHUNK)
        def _():
            @pl.when(i >= 1)
            def _():
                pltpu.make_async_copy(rows_v.at[pl.ds(noff, CH)],
                                      z_sh.at[dst_v.at[i - 1]], sem_s).wait()
            pltpu.async_copy(table_sh.at[src_v.at[i + 3]],
                             rows_v.at[pl.ds(noff, CH)], sem_g)

        return carry

    lax.fori_loop(0, NCHUNK, chunk, 0)
    for t in range(4):
        pltpu.make_async_copy(rows_v.at[pl.ds(0, CH)],
                              z_sh.at[dst_v.at[NCHUNK - 4 + t]], sem_s).wait()
    plsc.subcore_barrier()
    pltpu.sync_copy(z_sh.at[pl.ds(rs, RPT)], buf_v)
    pltpu.sync_copy(buf_v, z_out.at[c, pl.ds(rs, RPT)])


def _prop_call(hws, srcg, dstg, zeros32):
    fn = pl.kernel(
        _prop_body,
        out_type=jax.ShapeDtypeStruct((NC, NP, HH), _f32),
        mesh=_sc_mesh(),
        compiler_params=_sc_params(),
        scratch_types=[
            pltpu.VMEM((NCHUNK, CH), _i32),
            pltpu.VMEM((NCHUNK, CH), _i32),
            pltpu.VMEM((4 * CH, HH), _f32),
            pltpu.VMEM((RPT, HH), _f32),
            pltpu.VMEM((RPT, HH), _f32),
            pltpu.SemaphoreType.DMA,
            pltpu.SemaphoreType.DMA,
            pltpu.SemaphoreType.DMA,
            pltpu.SemaphoreType.DMA,
            pltpu.SemaphoreType.DMA,
            pltpu.SemaphoreType.DMA,
            pltpu.VMEM_SHARED((NP, HH), _f32),
            pltpu.VMEM_SHARED((NP, HH), _f32),
        ],
    )
    return fn(hws, srcg, dstg, zeros32)


_TCW = 320


def _tc1a_body(xw_ref, wb_ref, hw_ref):
    hw_ref[...] = jnp.dot(xw_ref[...], wb_ref[...],
                          preferred_element_type=_f32)


def _tc1a_call(xw, W1b):
    return pl.pallas_call(
        _tc1a_body,
        grid=(NW4 // _TCW,),
        in_specs=[
            pl.BlockSpec((_TCW, 4 * DD), lambda i: (i, 0)),
            pl.BlockSpec((4 * DD, 4 * HH), lambda i: (0, 0)),
        ],
        out_specs=pl.BlockSpec((_TCW, 4 * HH), lambda i: (i, 0)),
        out_shape=jax.ShapeDtypeStruct((NW4, 4 * HH), _f32),
    )(xw, W1b)


def _tc1b_body(degp_ref, hw_ref, disw_ref, hws_ref):
    deg = degp_ref[0] + degp_ref[1] + 1.0
    disw = lax.rsqrt(deg)
    disw_ref[...] = disw
    hws_ref[...] = hw_ref[...] * disw


def _tc1b_call(degpw, hw1):
    return pl.pallas_call(
        _tc1b_body,
        grid=(NW4 // _TCW,),
        in_specs=[
            pl.BlockSpec((NC, _TCW, 4 * HH), lambda i: (0, i, 0)),
            pl.BlockSpec((_TCW, 4 * HH), lambda i: (i, 0)),
        ],
        out_specs=(
            pl.BlockSpec((_TCW, 4 * HH), lambda i: (i, 0)),
            pl.BlockSpec((_TCW, 4 * HH), lambda i: (i, 0)),
        ),
        out_shape=(
            jax.ShapeDtypeStruct((NW4, 4 * HH), _f32),
            jax.ShapeDtypeStruct((NW4, 4 * HH), _f32),
        ),
    )(degpw, hw1)


def _tc_mid_body(zp_ref, hws_ref, disw_ref, b_ref, wb_ref, out_ref):
    agg = zp_ref[0] + zp_ref[1] + hws_ref[...]
    disw = disw_ref[...]
    h = jnp.maximum(agg * disw + b_ref[...], 0.0)
    out_ref[...] = jnp.dot(h, wb_ref[...], preferred_element_type=_f32) * disw


def _tc_mid_call(zpw, hws, disw, bw, Wb):
    return pl.pallas_call(
        _tc_mid_body,
        grid=(NW4 // _TCW,),
        in_specs=[
            pl.BlockSpec((NC, _TCW, 4 * HH), lambda i: (0, i, 0)),
            pl.BlockSpec((_TCW, 4 * HH), lambda i: (i, 0)),
            pl.BlockSpec((_TCW, 4 * HH), lambda i: (i, 0)),
            pl.BlockSpec((1, 4 * HH), lambda i: (0, 0)),
            pl.BlockSpec((4 * HH, 4 * HH), lambda i: (0, 0)),
        ],
        out_specs=pl.BlockSpec((_TCW, 4 * HH), lambda i: (i, 0)),
        out_shape=jax.ShapeDtypeStruct((NW4, 4 * HH), _f32),
    )(zpw, hws, disw, bw, Wb)


def _tc4_body(zp_ref, hws_ref, disw_ref, b3_ref, m1w_ref, m1b_ref, m2w_ref,
              m2b_ref, out_ref):
    agg = zp_ref[0] + zp_ref[1] + hws_ref[...]
    h = jnp.maximum(agg * disw_ref[...] + b3_ref[...], 0.0)
    h = jnp.maximum(jnp.dot(h, m1w_ref[...], preferred_element_type=_f32)
                    + m1b_ref[...], 0.0)
    h = jnp.maximum(jnp.dot(h, m2w_ref[...], preferred_element_type=_f32)
                    + m2b_ref[...], 0.0)
    out_ref[...] = h


def _tc4_call(zpw, hws, disw, b3w, M1Wb, M1bw, M2Wb, M2bw):
    return pl.pallas_call(
        _tc4_body,
        grid=(NW4 // _TCW,),
        in_specs=[
            pl.BlockSpec((NC, _TCW, 4 * HH), lambda i: (0, i, 0)),
            pl.BlockSpec((_TCW, 4 * HH), lambda i: (i, 0)),
            pl.BlockSpec((_TCW, 4 * HH), lambda i: (i, 0)),
            pl.BlockSpec((1, 4 * HH), lambda i: (0, 0)),
            pl.BlockSpec((4 * HH, 4 * 64), lambda i: (0, 0)),
            pl.BlockSpec((1, 4 * 64), lambda i: (0, 0)),
            pl.BlockSpec((4 * 64, 4 * HH), lambda i: (0, 0)),
            pl.BlockSpec((1, 4 * HH), lambda i: (0, 0)),
        ],
        out_specs=pl.BlockSpec((_TCW, 4 * HH), lambda i: (i, 0)),
        out_shape=jax.ShapeDtypeStruct((NW4, 4 * HH), _f32),
    )(zpw, hws, disw, b3w, M1Wb, M1bw, M2Wb, M2bw)


def _blockdiag4(W):
    return jnp.kron(jnp.eye(4, dtype=W.dtype), W)


def kernel(x, edge_index, W1, b1, W2, b2, W3, b3, M1W, M1b, M2W, M2b):
    eig = edge_index.reshape(2, NW, NCHUNK, CH)
    srcg = eig[0]
    dstg = eig[1]
    ones32 = jnp.ones((CH, HH), _f32)
    zeros32 = jnp.zeros((RPT, HH), _f32)
    xw = jnp.pad(x, ((0, NP - NN), (0, 0))).reshape(NW4, 4 * DD)

    W1b = _blockdiag4(W1)
    W2b = _blockdiag4(W2)
    W3b = _blockdiag4(W3)
    M1Wb = _blockdiag4(M1W)
    M2Wb = _blockdiag4(M2W)
    b1w = jnp.tile(b1, 4).reshape(1, 4 * HH)
    b2w = jnp.tile(b2, 4).reshape(1, 4 * HH)
    b3w = jnp.tile(b3, 4).reshape(1, 4 * HH)
    M1bw = jnp.tile(M1b, 4).reshape(1, 4 * 64)
    M2bw = jnp.tile(M2b, 4).reshape(1, 4 * HH)

    hw1 = _tc1a_call(xw, W1b)
    degp = _deg_call(dstg, ones32, zeros32)
    degpw = degp.reshape(NC, NW4, 4 * HH)
    disw, hws1 = _tc1b_call(degpw, hw1)

    hws1n = hws1.reshape(NP, HH)
    z1 = _prop_call(hws1n, srcg, dstg, zeros32)
    hws2 = _tc_mid_call(z1.reshape(NC, NW4, 4 * HH), hws1, disw, b1w, W2b)

    hws2n = hws2.reshape(NP, HH)
    z2 = _prop_call(hws2n, srcg, dstg, zeros32)
    hws3 = _tc_mid_call(z2.reshape(NC, NW4, 4 * HH), hws2, disw, b2w, W3b)

    hws3n = hws3.reshape(NP, HH)
    z3 = _prop_call(hws3n, srcg, dstg, zeros32)
    out = _tc4_call(z3.reshape(NC, NW4, 4 * HH), hws3, disw, b3w,
                    M1Wb, M1bw, M2Wb, M2bw)
    return out.reshape(NP, HH)[:NN]
